# baseline (speedup 1.0000x reference)
import jax
import jax.numpy as jnp
from jax.experimental import pallas as pl
from jax.experimental.pallas import tpu as pltpu

_M = 0.25
_GAMMA = 64.0
_MG = _M * _GAMMA
_SG = (1.0 - _M) * _GAMMA
_NEG = -1e30

_B = 1024
_C = 100000
_RB = 128
_NRB = _B // _RB
_W = 8192
_K = (_C + _W - 1) // _W
_NCH = _W // 128


def _body(inp_ref, lab_ref, out_ref, acc_ref, mx_ref, g_ref, tot_ref):
    rb = pl.program_id(0)
    k = pl.program_id(1)

    @pl.when(jnp.logical_and(rb == 0, k == 0))
    def _zero_tot():
        tot_ref[0, 0] = 0.0

    @pl.when(k == 0)
    def _init():
        acc_ref[...] = jnp.zeros_like(acc_ref)
        mx_ref[...] = jnp.zeros_like(mx_ref)
        g_ref[...] = jnp.zeros_like(g_ref)

    lab = lab_ref[...]
    base = k * _W
    lane = jax.lax.broadcasted_iota(jnp.int32, (1, 128), 1)

    def sweep(maskpad):
        a = acc_ref[...]
        m = mx_ref[...]
        g = g_ref[...]
        for j in range(_NCH):
            xc = inp_ref[:, j * 128:(j + 1) * 128]
            cols = lane + (base + j * 128)
            is_lab = cols == lab
            lg = jnp.maximum(xc + _M, 0.0) * (xc * _GAMMA - _MG)
            if maskpad:
                bad = jnp.logical_or(is_lab, cols >= _C)
            else:
                bad = is_lab
            lg = jnp.where(bad, _NEG, lg)
            m_new = jnp.maximum(m, lg)
            a = a * jnp.exp(m - m_new) + jnp.exp(lg - m_new)
            m = m_new
            g = g + jnp.where(is_lab, xc, 0.0)
        acc_ref[...] = a
        mx_ref[...] = m
        g_ref[...] = g

    @pl.when(k < _K - 1)
    def _hot():
        sweep(False)

    @pl.when(k == _K - 1)
    def _last():
        sweep(True)
        gl = jnp.sum(g_ref[...], axis=1, keepdims=True)
        tl = jnp.maximum(1.0 + _M - gl, 0.0) * (gl * _GAMMA - _SG)
        mrow = jnp.max(mx_ref[...], axis=1, keepdims=True)
        s = jnp.sum(acc_ref[...] * jnp.exp(mx_ref[...] - mrow),
                    axis=1, keepdims=True)
        m_f = jnp.maximum(mrow, tl)
        lse = m_f + jnp.log(s * jnp.exp(mrow - m_f) + jnp.exp(tl - m_f))
        tot = tot_ref[0, 0] + jnp.sum(lse - tl)
        tot_ref[0, 0] = tot

        @pl.when(rb == _NRB - 1)
        def _out():
            out_ref[0, 0] = tot * (1.0 / _B)


@jax.jit
def kernel(inp, label):
    lab2 = label.reshape(_B, 1)
    out = pl.pallas_call(
        _body,
        grid=(_NRB, _K),
        in_specs=[
            pl.BlockSpec((_RB, _W), lambda rb, k: (rb, k)),
            pl.BlockSpec((_RB, 1), lambda rb, k: (rb, 0)),
        ],
        out_specs=pl.BlockSpec(
            (1, 1), lambda rb, k: (0, 0), memory_space=pltpu.SMEM),
        out_shape=jax.ShapeDtypeStruct((1, 1), jnp.float32),
        scratch_shapes=[
            pltpu.VMEM((_RB, 128), jnp.float32),
            pltpu.VMEM((_RB, 128), jnp.float32),
            pltpu.VMEM((_RB, 128), jnp.float32),
            pltpu.SMEM((1, 1), jnp.float32),
        ],
        compiler_params=pltpu.CompilerParams(
            dimension_semantics=("arbitrary", "arbitrary"),
        ),
    )(inp, lab2)
    return out[0, 0]


# speedup vs baseline: 1.0872x; 1.0872x over previous
import jax
import jax.numpy as jnp
from jax.experimental import pallas as pl
from jax.experimental.pallas import tpu as pltpu
from jax.experimental.pallas import tpu_sc as plsc

_M = 0.25
_GAMMA = 64.0
_MG = _M * _GAMMA
_SG = (1.0 - _M) * _GAMMA
_NEG = -1e30

_B = 1024
_C = 100000
_RB = 128
_NRB = _B // _RB
_W = 8192
_K = (_C + _W - 1) // _W
_NCH = _W // 128

_NW = 32
_RPW = _B // _NW


def _sc_gather_body(inp_hbm, lab_hbm, out_hbm, lab_v, tile_v, chunk_v):
    c = jax.lax.axis_index("c")
    s = jax.lax.axis_index("s")
    wid = s * 2 + c
    base = wid * _RPW
    pltpu.sync_copy(lab_hbm.at[pl.ds(base, _RPW)], lab_v)
    lane16 = jax.lax.iota(jnp.int32, 16)
    for r in range(_RPW):
        vec = lab_v[pl.ds((r // 16) * 16, 16)]
        lab_r = jnp.max(jnp.where(lane16 == (r % 16), vec, -1))
        col0 = pl.multiple_of(jax.lax.bitwise_and(lab_r, -128), 128)
        seg = jax.lax.bitwise_and(lab_r, 112)
        pltpu.sync_copy(
            inp_hbm.at[pl.ds(base + (r // 8) * 8, 8), pl.ds(col0, 128)],
            tile_v)
        chunk_v[r, :] = tile_v[r % 8, pl.ds(seg, 16)]
    pltpu.sync_copy(chunk_v, out_hbm.at[pl.ds(base, _RPW)])


def _sc_gather(inp, label):
    return pl.kernel(
        _sc_gather_body,
        out_type=jax.ShapeDtypeStruct((_B, 16), jnp.float32),
        mesh=plsc.VectorSubcoreMesh(core_axis_name="c", subcore_axis_name="s"),
        compiler_params=pltpu.CompilerParams(needs_layout_passes=False),
        scratch_types=[
            pltpu.VMEM((_RPW,), jnp.int32),
            pltpu.VMEM((8, 128), jnp.float32),
            pltpu.VMEM((_RPW, 16), jnp.float32),
        ],
    )(inp, label)


def _stream_body(inp_ref, acc_ref, mx_ref):
    k = pl.program_id(1)

    @pl.when(k == 0)
    def _init():
        acc_ref[...] = jnp.zeros_like(acc_ref)
        mx_ref[...] = jnp.zeros_like(mx_ref)

    def sweep(maskpad):
        a = acc_ref[...]
        m = mx_ref[...]
        if maskpad:
            base = k * _W
            lane = jax.lax.broadcasted_iota(jnp.int32, (1, 128), 1)
        for j in range(_NCH):
            xc = inp_ref[:, j * 128:(j + 1) * 128]
            lg = jnp.maximum(xc + _M, 0.0) * (xc * _GAMMA - _MG)
            if maskpad:
                cols = lane + (base + j * 128)
                lg = jnp.where(cols >= _C, _NEG, lg)
            m_new = jnp.maximum(m, lg)
            a = a * jnp.exp(m - m_new) + jnp.exp(lg - m_new)
            m = m_new
        acc_ref[...] = a
        mx_ref[...] = m

    @pl.when(k < _K - 1)
    def _hot():
        sweep(False)

    @pl.when(k == _K - 1)
    def _last():
        sweep(True)


def _stream(inp):
    return pl.pallas_call(
        _stream_body,
        grid=(_NRB, _K),
        in_specs=[
            pl.BlockSpec((_RB, _W), lambda rb, k: (rb, k)),
        ],
        out_specs=[
            pl.BlockSpec((_RB, 128), lambda rb, k: (rb, 0)),
            pl.BlockSpec((_RB, 128), lambda rb, k: (rb, 0)),
        ],
        out_shape=[
            jax.ShapeDtypeStruct((_B, 128), jnp.float32),
            jax.ShapeDtypeStruct((_B, 128), jnp.float32),
        ],
        compiler_params=pltpu.CompilerParams(
            dimension_semantics=("arbitrary", "arbitrary"),
        ),
    )(inp)


def _combine_body(acc_ref, mx_ref, chunk_ref, lab_ref, out_ref):
    lab = lab_ref[...]
    lane16 = jax.lax.broadcasted_iota(jnp.int32, (1, 16), 1)
    sel = lane16 == jax.lax.bitwise_and(lab, 15)
    g = jnp.sum(jnp.where(sel, chunk_ref[...], 0.0), axis=1,
                keepdims=True)
    wrong = jnp.maximum(g + _M, 0.0) * (g * _GAMMA - _MG)
    tl = jnp.maximum(1.0 + _M - g, 0.0) * (g * _GAMMA - _SG)
    mx = mx_ref[...]
    mrow = jnp.max(mx, axis=1, keepdims=True)
    srow = jnp.sum(acc_ref[...] * jnp.exp(mx - mrow), axis=1, keepdims=True)
    s_corr = jnp.maximum(srow - jnp.exp(wrong - mrow), 1e-20)
    m_f = jnp.maximum(mrow, tl)
    lse = m_f + jnp.log(s_corr * jnp.exp(mrow - m_f) + jnp.exp(tl - m_f))
    out_ref[0, 0] = jnp.sum(lse - tl) * (1.0 / _B)


def _combine(acc, mx, chunks, lab2):
    return pl.pallas_call(
        _combine_body,
        out_specs=pl.BlockSpec(memory_space=pltpu.SMEM),
        out_shape=jax.ShapeDtypeStruct((1, 1), jnp.float32),
    )(acc, mx, chunks, lab2)


@jax.jit
def kernel(inp, label):
    chunks = _sc_gather(inp, label)
    acc, mx = _stream(inp)
    out = _combine(acc, mx, chunks, label.reshape(_B, 1))
    return out[0, 0]


# speedup vs baseline: 1.1330x; 1.0421x over previous
import jax
import jax.numpy as jnp
from jax.experimental import pallas as pl
from jax.experimental.pallas import tpu as pltpu
from jax.experimental.pallas import tpu_sc as plsc

_M = 0.25
_GAMMA = 64.0
_MG = _M * _GAMMA
_SG = (1.0 - _M) * _GAMMA
_NEG = -1e30

_B = 1024
_C = 100000
_RB = 128
_NRB = _B // _RB
_W = 8192
_K = (_C + _W - 1) // _W
_NCH = _W // 128

_NW = 32
_RPW = _B // _NW


def _sc_gather_body(inp_hbm, lab_hbm, out_hbm, lab_v, tile_v, chunk_v):
    c = jax.lax.axis_index("c")
    s = jax.lax.axis_index("s")
    wid = s * 2 + c
    base = wid * _RPW
    pltpu.sync_copy(lab_hbm.at[pl.ds(base, _RPW)], lab_v)
    lane16 = jax.lax.iota(jnp.int32, 16)
    for r in range(_RPW):
        vec = lab_v[pl.ds((r // 16) * 16, 16)]
        lab_r = jnp.max(jnp.where(lane16 == (r % 16), vec, -1))
        col0 = pl.multiple_of(jax.lax.bitwise_and(lab_r, -128), 128)
        seg = jax.lax.bitwise_and(lab_r, 112)
        pltpu.sync_copy(
            inp_hbm.at[pl.ds(base + (r // 8) * 8, 8), pl.ds(col0, 128)],
            tile_v)
        chunk_v[r, :] = tile_v[r % 8, pl.ds(seg, 16)]
    pltpu.sync_copy(chunk_v, out_hbm.at[pl.ds(base, _RPW)])


def _sc_gather(inp, label):
    return pl.kernel(
        _sc_gather_body,
        out_type=jax.ShapeDtypeStruct((_B, 16), jnp.float32),
        mesh=plsc.VectorSubcoreMesh(core_axis_name="c", subcore_axis_name="s"),
        compiler_params=pltpu.CompilerParams(needs_layout_passes=False),
        scratch_types=[
            pltpu.VMEM((_RPW,), jnp.int32),
            pltpu.VMEM((8, 128), jnp.float32),
            pltpu.VMEM((_RPW, 16), jnp.float32),
        ],
    )(inp, label)


def _stream_body(inp_ref, acc_ref, mx_ref, lg_ref):
    k = pl.program_id(1)

    @pl.when(k == 0)
    def _init():
        acc_ref[...] = jnp.zeros_like(acc_ref)
        mx_ref[...] = jnp.zeros_like(mx_ref)

    def sweep(maskpad):
        a = acc_ref[...]
        m = mx_ref[...]
        bm = m
        if maskpad:
            base = k * _W
            lane = jax.lax.broadcasted_iota(jnp.int32, (1, 128), 1)
        for j in range(_NCH):
            xc = inp_ref[:, j * 128:(j + 1) * 128]
            lg = jnp.maximum(xc + _M, 0.0) * (xc * _GAMMA - _MG)
            if maskpad:
                cols = lane + (base + j * 128)
                lg = jnp.where(cols >= _C, _NEG, lg)
            lg_ref[:, j * 128:(j + 1) * 128] = lg
            bm = jnp.maximum(bm, lg)
        a = a * jnp.exp(m - bm)
        for j in range(_NCH):
            a = a + jnp.exp(lg_ref[:, j * 128:(j + 1) * 128] - bm)
        acc_ref[...] = a
        mx_ref[...] = bm

    @pl.when(k < _K - 1)
    def _hot():
        sweep(False)

    @pl.when(k == _K - 1)
    def _last():
        sweep(True)


def _stream(inp):
    return pl.pallas_call(
        _stream_body,
        grid=(_NRB, _K),
        in_specs=[
            pl.BlockSpec((_RB, _W), lambda rb, k: (rb, k)),
        ],
        out_specs=[
            pl.BlockSpec((_RB, 128), lambda rb, k: (rb, 0)),
            pl.BlockSpec((_RB, 128), lambda rb, k: (rb, 0)),
        ],
        out_shape=[
            jax.ShapeDtypeStruct((_B, 128), jnp.float32),
            jax.ShapeDtypeStruct((_B, 128), jnp.float32),
        ],
        scratch_shapes=[
            pltpu.VMEM((_RB, _W), jnp.float32),
        ],
        compiler_params=pltpu.CompilerParams(
            dimension_semantics=("arbitrary", "arbitrary"),
        ),
    )(inp)


def _combine_body(acc_ref, mx_ref, chunk_ref, lab_ref, out_ref):
    lab = lab_ref[...]
    lane16 = jax.lax.broadcasted_iota(jnp.int32, (1, 16), 1)
    sel = lane16 == jax.lax.bitwise_and(lab, 15)
    g = jnp.sum(jnp.where(sel, chunk_ref[...], 0.0), axis=1,
                keepdims=True)
    wrong = jnp.maximum(g + _M, 0.0) * (g * _GAMMA - _MG)
    tl = jnp.maximum(1.0 + _M - g, 0.0) * (g * _GAMMA - _SG)
    mx = mx_ref[...]
    mrow = jnp.max(mx, axis=1, keepdims=True)
    srow = jnp.sum(acc_ref[...] * jnp.exp(mx - mrow), axis=1, keepdims=True)
    s_corr = jnp.maximum(srow - jnp.exp(wrong - mrow), 1e-20)
    m_f = jnp.maximum(mrow, tl)
    lse = m_f + jnp.log(s_corr * jnp.exp(mrow - m_f) + jnp.exp(tl - m_f))
    out_ref[0, 0] = jnp.sum(lse - tl) * (1.0 / _B)


def _combine(acc, mx, chunks, lab2):
    return pl.pallas_call(
        _combine_body,
        out_specs=pl.BlockSpec(memory_space=pltpu.SMEM),
        out_shape=jax.ShapeDtypeStruct((1, 1), jnp.float32),
    )(acc, mx, chunks, lab2)


@jax.jit
def kernel(inp, label):
    chunks = _sc_gather(inp, label)
    acc, mx = _stream(inp)
    out = _combine(acc, mx, chunks, label.reshape(_B, 1))
    return out[0, 0]


# speedup vs baseline: 1.1853x; 1.0462x over previous
import jax
import jax.numpy as jnp
from jax.experimental import pallas as pl
from jax.experimental.pallas import tpu as pltpu
from jax.experimental.pallas import tpu_sc as plsc

_M = 0.25
_GAMMA = 64.0
_MG = _M * _GAMMA
_SG = (1.0 - _M) * _GAMMA
_NEG = -1e30

_B = 1024
_C = 100000
_RB = 128
_NRB = _B // _RB
_W = 12544
_K = (_C + _W - 1) // _W
_NCH = _W // 128

_NW = 32
_RPW = _B // _NW


def _sc_gather_body(inp_hbm, lab_hbm, out_hbm, lab_v, tile_v, chunk_v):
    c = jax.lax.axis_index("c")
    s = jax.lax.axis_index("s")
    wid = s * 2 + c
    base = wid * _RPW
    pltpu.sync_copy(lab_hbm.at[pl.ds(base, _RPW)], lab_v)
    lane16 = jax.lax.iota(jnp.int32, 16)
    for r in range(_RPW):
        vec = lab_v[pl.ds((r // 16) * 16, 16)]
        lab_r = jnp.max(jnp.where(lane16 == (r % 16), vec, -1))
        col0 = pl.multiple_of(jax.lax.bitwise_and(lab_r, -128), 128)
        seg = jax.lax.bitwise_and(lab_r, 112)
        pltpu.sync_copy(
            inp_hbm.at[pl.ds(base + (r // 8) * 8, 8), pl.ds(col0, 128)],
            tile_v)
        chunk_v[r, :] = tile_v[r % 8, pl.ds(seg, 16)]
    pltpu.sync_copy(chunk_v, out_hbm.at[pl.ds(base, _RPW)])


def _sc_gather(inp, label):
    return pl.kernel(
        _sc_gather_body,
        out_type=jax.ShapeDtypeStruct((_B, 16), jnp.float32),
        mesh=plsc.VectorSubcoreMesh(core_axis_name="c", subcore_axis_name="s"),
        compiler_params=pltpu.CompilerParams(needs_layout_passes=False),
        scratch_types=[
            pltpu.VMEM((_RPW,), jnp.int32),
            pltpu.VMEM((8, 128), jnp.float32),
            pltpu.VMEM((_RPW, 16), jnp.float32),
        ],
    )(inp, label)


def _stream_body(inp_ref, acc_ref, mx_ref, lg_ref):
    k = pl.program_id(1)

    @pl.when(k == 0)
    def _init():
        acc_ref[...] = jnp.zeros_like(acc_ref)
        mx_ref[...] = jnp.zeros_like(mx_ref)

    def sweep(maskpad):
        a = acc_ref[...]
        m = mx_ref[...]
        bm = m
        if maskpad:
            base = k * _W
            lane = jax.lax.broadcasted_iota(jnp.int32, (1, 128), 1)
        for j in range(_NCH):
            xc = inp_ref[:, j * 128:(j + 1) * 128]
            lg = jnp.maximum(xc + _M, 0.0) * (xc * _GAMMA - _MG)
            if maskpad:
                cols = lane + (base + j * 128)
                lg = jnp.where(cols >= _C, _NEG, lg)
            lg_ref[:, j * 128:(j + 1) * 128] = lg
            bm = jnp.maximum(bm, lg)
        a = a * jnp.exp(m - bm)
        for j in range(_NCH):
            a = a + jnp.exp(lg_ref[:, j * 128:(j + 1) * 128] - bm)
        acc_ref[...] = a
        mx_ref[...] = bm

    @pl.when(k < _K - 1)
    def _hot():
        sweep(False)

    @pl.when(k == _K - 1)
    def _last():
        sweep(True)


def _stream(inp):
    return pl.pallas_call(
        _stream_body,
        grid=(_NRB, _K),
        in_specs=[
            pl.BlockSpec((_RB, _W), lambda rb, k: (rb, k)),
        ],
        out_specs=[
            pl.BlockSpec((_RB, 128), lambda rb, k: (rb, 0)),
            pl.BlockSpec((_RB, 128), lambda rb, k: (rb, 0)),
        ],
        out_shape=[
            jax.ShapeDtypeStruct((_B, 128), jnp.float32),
            jax.ShapeDtypeStruct((_B, 128), jnp.float32),
        ],
        scratch_shapes=[
            pltpu.VMEM((_RB, _W), jnp.float32),
        ],
        compiler_params=pltpu.CompilerParams(
            dimension_semantics=("arbitrary", "arbitrary"),
        ),
    )(inp)


def _combine_body(acc_ref, mx_ref, chunk_ref, lab_ref, out_ref):
    lab = lab_ref[...]
    lane16 = jax.lax.broadcasted_iota(jnp.int32, (1, 16), 1)
    sel = lane16 == jax.lax.bitwise_and(lab, 15)
    g = jnp.sum(jnp.where(sel, chunk_ref[...], 0.0), axis=1,
                keepdims=True)
    wrong = jnp.maximum(g + _M, 0.0) * (g * _GAMMA - _MG)
    tl = jnp.maximum(1.0 + _M - g, 0.0) * (g * _GAMMA - _SG)
    mx = mx_ref[...]
    mrow = jnp.max(mx, axis=1, keepdims=True)
    srow = jnp.sum(acc_ref[...] * jnp.exp(mx - mrow), axis=1, keepdims=True)
    s_corr = jnp.maximum(srow - jnp.exp(wrong - mrow), 1e-20)
    m_f = jnp.maximum(mrow, tl)
    lse = m_f + jnp.log(s_corr * jnp.exp(mrow - m_f) + jnp.exp(tl - m_f))
    out_ref[0, 0] = jnp.sum(lse - tl) * (1.0 / _B)


def _combine(acc, mx, chunks, lab2):
    return pl.pallas_call(
        _combine_body,
        out_specs=pl.BlockSpec(memory_space=pltpu.SMEM),
        out_shape=jax.ShapeDtypeStruct((1, 1), jnp.float32),
    )(acc, mx, chunks, lab2)


@jax.jit
def kernel(inp, label):
    chunks = _sc_gather(inp, label)
    acc, mx = _stream(inp)
    out = _combine(acc, mx, chunks, label.reshape(_B, 1))
    return out[0, 0]


# speedup vs baseline: 3.0920x; 2.6085x over previous
import jax
import jax.numpy as jnp
from jax.experimental import pallas as pl
from jax.experimental.pallas import tpu as pltpu
from jax.experimental.pallas import tpu_sc as plsc

_M = 0.25
_GAMMA = 64.0
_MG = _M * _GAMMA
_SG = (1.0 - _M) * _GAMMA
_NEG = -1e30

_B = 1024
_C = 100000
_WC = 2048
_K = (_C + _WC - 1) // _WC
_SL = 16
_NSL = _WC // _SL

_NW = 32
_RPW = _B // _NW


def _sc_gather_body(xt_hbm, lab_hbm, out_hbm, lab_v, tile_v, g_v):
    c = jax.lax.axis_index("c")
    s = jax.lax.axis_index("s")
    wid = s * 2 + c
    base = wid * _RPW
    lanec = (wid % 4) * _RPW
    col0 = (wid // 4) * 128
    pltpu.sync_copy(lab_hbm.at[pl.ds(base, _RPW)], lab_v)
    lane16 = jax.lax.iota(jnp.int32, 16)
    ones16 = jnp.full((16,), 1, jnp.int32)
    mask0 = lane16 == 0
    for r in range(_RPW):
        vec = lab_v[pl.ds((r // 16) * 16, 16)]
        lab_r = jnp.max(jnp.where(lane16 == (r % 16), vec, -1))
        row0 = pl.multiple_of(jax.lax.bitwise_and(lab_r, -8), 8)
        sub = jax.lax.bitwise_and(lab_r, 7)
        pltpu.sync_copy(xt_hbm.at[pl.ds(row0, 8), pl.ds(col0, 128)],
                        tile_v)
        g16 = plsc.load_gather(tile_v, [ones16 * sub,
                                        ones16 * (lanec + r)])
        plsc.store_scatter(g_v, [ones16 * r], g16, mask=mask0)
    pltpu.sync_copy(g_v, out_hbm.at[pl.ds(base, _RPW)])


def _sc_gather(xt, label):
    return pl.kernel(
        _sc_gather_body,
        out_type=jax.ShapeDtypeStruct((_B,), jnp.float32),
        mesh=plsc.VectorSubcoreMesh(core_axis_name="c", subcore_axis_name="s"),
        compiler_params=pltpu.CompilerParams(needs_layout_passes=False),
        scratch_types=[
            pltpu.VMEM((_RPW,), jnp.int32),
            pltpu.VMEM((8, 128), jnp.float32),
            pltpu.VMEM((_RPW,), jnp.float32),
        ],
    )(xt, label)


def _stream_body(xt_ref, acc_ref, mx_ref, lg_ref):
    k = pl.program_id(0)

    @pl.when(k == 0)
    def _init():
        acc_ref[...] = jnp.zeros_like(acc_ref)
        mx_ref[...] = jnp.zeros_like(mx_ref)

    def sweep(maskpad):
        m = mx_ref[...]
        bm = m
        if maskpad:
            base = k * _WC
            riota = jax.lax.broadcasted_iota(jnp.int32, (_SL, 1), 0)
        for j in range(_NSL):
            xc = xt_ref[j * _SL:(j + 1) * _SL, :]
            lg = jnp.maximum(xc + _M, 0.0) * (xc * _GAMMA - _MG)
            if maskpad:
                rows = riota + (base + j * _SL)
                lg = jnp.where(rows >= _C, _NEG, lg)
            lg_ref[j * _SL:(j + 1) * _SL, :] = lg
            bm = jnp.maximum(bm, lg)
        a = acc_ref[...] * jnp.exp(m - bm)
        for j in range(_NSL):
            a = a + jnp.exp(lg_ref[j * _SL:(j + 1) * _SL, :] - bm)
        acc_ref[...] = a
        mx_ref[...] = bm

    @pl.when(k < _K - 1)
    def _hot():
        sweep(False)

    @pl.when(k == _K - 1)
    def _last():
        sweep(True)


def _stream(xt):
    return pl.pallas_call(
        _stream_body,
        grid=(_K,),
        in_specs=[
            pl.BlockSpec((_WC, _B), lambda k: (k, 0)),
        ],
        out_specs=[
            pl.BlockSpec((_SL, _B), lambda k: (0, 0)),
            pl.BlockSpec((_SL, _B), lambda k: (0, 0)),
        ],
        out_shape=[
            jax.ShapeDtypeStruct((_SL, _B), jnp.float32),
            jax.ShapeDtypeStruct((_SL, _B), jnp.float32),
        ],
        scratch_shapes=[
            pltpu.VMEM((_WC, _B), jnp.float32),
        ],
        compiler_params=pltpu.CompilerParams(
            dimension_semantics=("arbitrary",),
        ),
    )(xt)


def _combine_body(acc_ref, mx_ref, g_ref, out_ref):
    g = g_ref[...]
    wrong = jnp.maximum(g + _M, 0.0) * (g * _GAMMA - _MG)
    tl = jnp.maximum(1.0 + _M - g, 0.0) * (g * _GAMMA - _SG)
    mx = mx_ref[...]
    mrow = jnp.max(mx, axis=0, keepdims=True)
    srow = jnp.sum(acc_ref[...] * jnp.exp(mx - mrow), axis=0, keepdims=True)
    s_corr = jnp.maximum(srow - jnp.exp(wrong - mrow), 1e-20)
    m_f = jnp.maximum(mrow, tl)
    lse = m_f + jnp.log(s_corr * jnp.exp(mrow - m_f) + jnp.exp(tl - m_f))
    out_ref[0, 0] = jnp.sum(lse - tl) * (1.0 / _B)


def _combine(acc, mx, g_row):
    return pl.pallas_call(
        _combine_body,
        out_specs=pl.BlockSpec(memory_space=pltpu.SMEM),
        out_shape=jax.ShapeDtypeStruct((1, 1), jnp.float32),
    )(acc, mx, g_row)


@jax.jit
def kernel(inp, label):
    xt = inp.T
    g = _sc_gather(xt, label)
    acc, mx = _stream(xt)
    out = _combine(acc, mx, g.reshape(1, _B))
    return out[0, 0]
